# Initial kernel scaffold; baseline (speedup 1.0000x reference)
import jax
import jax.numpy as jnp
from jax.experimental import pallas as pl


def kernel(x_s, edge_index_s, edge_type_s, batch_s, x_g, edge_index_g, edge_type_g, batch_g, depth, emb_s, comp1_s, basis1_s, root1_s, bias1_s, comp2_s, basis2_s, root2_s, bias2_s, emb_g, comp1_g, basis1_g, root1_g, bias1_g, comp2_g, basis2_g, root2_g, bias2_g, W_bil, b_bil, depth_emb, W1, b1, W2, b2):
    raise NotImplementedError("write your pallas kernel here")


# speedup vs baseline: 15.0293x; 15.0293x over previous
import functools

import jax
import jax.numpy as jnp
from jax import lax
from jax.experimental import pallas as pl
from jax.experimental.pallas import tpu as pltpu
from jax.experimental.pallas import tpu_sc as plsc

N = 10000
NP = 10240
E = 320000
EP = 321536
R = 8
G = 16
EMB = 64
EMBP = 128
HID = 128
CROSS = 32
VOC = 512
DE = 8
NT = 16
PT = EP // NT
CT = PT // 128
KT = R * NP
KR = KT // 128

_f32 = jnp.float32
_i32 = jnp.int32


@functools.cache
def _mesh():
    return plsc.VectorSubcoreMesh(core_axis_name="c", subcore_axis_name="s",
                                  num_cores=2, num_subcores=NT)


def _pass_a(ids_h, src_h, dst_h, et_h, emb_h,
            x_out, gk_out, sk_out, cnt_out,
            idxb, xrows, srcb, dstb, etb, gkb, skb, cntb):
    cid = lax.axis_index("c")
    t = lax.axis_index("s")
    ones16 = jnp.ones((16,), _f32)
    zeros16 = jnp.zeros((16,), _f32)

    def _zrow(r, carry):
        cntb[pl.ds(r * 16, 16)] = zeros16
        return carry
    lax.fori_loop(0, KT // 16, _zrow, 0)

    def _xchunk(c, carry):
        row0 = pl.multiple_of(t * 640 + c * 128, 128)
        pltpu.sync_copy(ids_h.at[cid, pl.ds(row0, 128)], idxb)
        for j in range(8):
            s = pl.ds(j * 16, 16)
            idxb[s] = idxb[s] + cid * N
        pltpu.sync_copy(emb_h.at[idxb], xrows)
        pltpu.sync_copy(xrows, x_out.at[cid, pl.ds(row0, 128)])
        return carry
    lax.fori_loop(0, 5, _xchunk, 0)

    def _echunk(c, carry):
        base = pl.multiple_of(t * PT + c * 128, 128)
        pltpu.sync_copy(src_h.at[cid, pl.ds(base, 128)], srcb)
        pltpu.sync_copy(dst_h.at[cid, pl.ds(base, 128)], dstb)
        pltpu.sync_copy(et_h.at[cid, pl.ds(base, 128)], etb)
        for j in range(8):
            s = pl.ds(j * 16, 16)
            skey = dstb[s] * 8 + etb[s]
            plsc.addupdate_scatter(cntb, [skey], ones16)
            skb[s] = skey
            gkey = etb[s] * NP + srcb[s] + cid * KT
            gkb[s] = jnp.bitwise_or(gkey, lax.shift_left(dstb[s], 18))
        pltpu.sync_copy(gkb, gk_out.at[cid, pl.ds(base, 128)])
        pltpu.sync_copy(skb, sk_out.at[cid, pl.ds(base, 128)])
        return carry
    lax.fori_loop(0, CT, _echunk, 0)

    pltpu.sync_copy(cntb, cnt_out.at[cid, t])


@functools.cache
def _pass_a_call():
    return pl.kernel(
        _pass_a,
        out_type=[
            jax.ShapeDtypeStruct((2, NP, EMBP), _f32),
            jax.ShapeDtypeStruct((2, EP), _i32),
            jax.ShapeDtypeStruct((2, EP), _i32),
            jax.ShapeDtypeStruct((2, NT, KT), _f32),
        ],
        mesh=_mesh(),
        scratch_types=[
            pltpu.VMEM((128,), _i32),
            pltpu.VMEM((128, EMBP), _f32),
            pltpu.VMEM((128,), _i32),
            pltpu.VMEM((128,), _i32),
            pltpu.VMEM((128,), _i32),
            pltpu.VMEM((128,), _i32),
            pltpu.VMEM((128,), _i32),
            pltpu.VMEM((KT,), _f32),
        ],
        compiler_params=pltpu.CompilerParams(needs_layout_passes=False),
    )


def _pass_a2(sk_h, inv_h, sc_out, skb, scb, invb):
    cid = lax.axis_index("c")
    t = lax.axis_index("s")
    pltpu.sync_copy(inv_h.at[cid], invb)

    def _chunk(c, carry):
        base = pl.multiple_of(t * PT + c * 128, 128)
        pltpu.sync_copy(sk_h.at[cid, pl.ds(base, 128)], skb)
        for j in range(8):
            s = pl.ds(j * 16, 16)
            scb[s] = plsc.load_gather(invb, [skb[s]])
        pltpu.sync_copy(scb, sc_out.at[cid, pl.ds(base, 128)])
        return carry
    lax.fori_loop(0, CT, _chunk, 0)


@functools.cache
def _pass_a2_call():
    return pl.kernel(
        _pass_a2,
        out_type=jax.ShapeDtypeStruct((2, EP), _f32),
        mesh=_mesh(),
        scratch_types=[
            pltpu.VMEM((128,), _i32),
            pltpu.VMEM((128,), _f32),
            pltpu.VMEM((KT,), _f32),
        ],
        compiler_params=pltpu.CompilerParams(needs_layout_passes=False),
    )


def _pass_b(h_flat, meta_h, acc_out,
            rows, cb, sb, gkb, dstb, scb, accs):
    cid = lax.axis_index("c")
    t = lax.axis_index("s")
    zeros16 = jnp.zeros((16,), _f32)

    def _zrow(r, carry):
        for j in range(8):
            rows[r, pl.ds(j * 16, 16)] = zeros16
        return carry
    lax.fori_loop(0, 128, _zrow, 0)
    for c in range(5):
        pltpu.sync_copy(rows, accs.at[pl.ds(t * 640 + c * 128, 128)])
    plsc.subcore_barrier()

    def _chunk(c, carry):
        base = pl.multiple_of(t * PT + c * 128, 128)
        pltpu.sync_copy(meta_h.at[cid, 0, pl.ds(base, 128)], cb)
        pltpu.sync_copy(meta_h.at[cid, 1, pl.ds(base, 128)], sb)
        for j in range(8):
            s = pl.ds(j * 16, 16)
            cv = cb[s]
            gkb[s] = jnp.bitwise_and(cv, (1 << 18) - 1)
            dstb[s] = lax.shift_right_logical(cv, 18)
            scb[s] = plsc.bitcast(sb[s], _f32)
        pltpu.sync_copy(h_flat.at[gkb], rows)

        def _edge(e, c2):
            sp = plsc.load_gather(scb, [jnp.full((16,), e, _i32)])
            for j in range(8):
                s = pl.ds(j * 16, 16)
                rows[e, s] = rows[e, s] * sp
            return c2
        lax.fori_loop(0, 128, _edge, 0)
        pltpu.sync_copy(rows, accs.at[dstb], add=True)
        return carry
    lax.fori_loop(0, CT, _chunk, 0)
    plsc.subcore_barrier()

    for c in range(5):
        r0 = t * 640 + c * 128
        pltpu.sync_copy(accs.at[pl.ds(r0, 128)],
                        acc_out.at[cid, pl.ds(r0, 128)])


@functools.cache
def _pass_b_call():
    return pl.kernel(
        _pass_b,
        out_type=jax.ShapeDtypeStruct((2, NP, HID), _f32),
        mesh=_mesh(),
        scratch_types=[
            pltpu.VMEM((128, HID), _f32),
            pltpu.VMEM((128,), _i32),
            pltpu.VMEM((128,), _i32),
            pltpu.VMEM((128,), _i32),
            pltpu.VMEM((128,), _i32),
            pltpu.VMEM((128,), _f32),
            pltpu.VMEM_SHARED((NP, HID), _f32),
        ],
        compiler_params=pltpu.CompilerParams(needs_layout_passes=False),
    )


def _dense1_body(x_ref, basis_ref, comp_ref, root_ref, bias_ref, cnt_ref,
                 h_ref, o_ref, inv_ref):
    e = pl.program_id(0)
    r = pl.program_id(1)
    x = x_ref[0]
    w = comp_ref[e, r, 0] * basis_ref[0, 0]
    for b in range(1, R):
        w = w + comp_ref[e, r, b] * basis_ref[0, b]
    h_ref[0, 0] = jnp.dot(x, w, preferred_element_type=_f32)

    @pl.when(r == 0)
    def _():
        o_ref[0] = (jnp.dot(x, root_ref[0], preferred_element_type=_f32)
                    + bias_ref[0, 0])
        cnt = jnp.sum(cnt_ref[0], axis=0)
        inv_ref[0] = 1.0 / jnp.maximum(cnt, 1.0)


def _dense1(x, basis, comp, root, bias, cntp):
    return pl.pallas_call(
        _dense1_body,
        grid=(2, R),
        in_specs=[
            pl.BlockSpec((1, NP, EMBP), lambda e, r: (e, 0, 0)),
            pl.BlockSpec((1, R, EMBP, HID), lambda e, r: (e, 0, 0, 0)),
            pl.BlockSpec(memory_space=pltpu.SMEM),
            pl.BlockSpec((1, EMBP, HID), lambda e, r: (e, 0, 0)),
            pl.BlockSpec((1, 1, HID), lambda e, r: (e, 0, 0)),
            pl.BlockSpec((1, NT, KR, 128), lambda e, r: (e, 0, 0, 0)),
        ],
        out_specs=[
            pl.BlockSpec((1, 1, NP, HID), lambda e, r: (e, r, 0, 0)),
            pl.BlockSpec((1, NP, HID), lambda e, r: (e, 0, 0)),
            pl.BlockSpec((1, KR, 128), lambda e, r: (e, 0, 0)),
        ],
        out_shape=[
            jax.ShapeDtypeStruct((2, R, NP, HID), _f32),
            jax.ShapeDtypeStruct((2, NP, HID), _f32),
            jax.ShapeDtypeStruct((2, KR, 128), _f32),
        ],
    )(x, basis, comp, root, bias, cntp)


def _dense2_body(o1_ref, a1_ref, basis_ref, comp_ref, root_ref, bias_ref,
                 h_ref, o_ref):
    e = pl.program_id(0)
    r = pl.program_id(1)
    x = jnp.maximum(o1_ref[0] + a1_ref[0], 0.0)
    w = comp_ref[e, r, 0] * basis_ref[0, 0]
    for b in range(1, R):
        w = w + comp_ref[e, r, b] * basis_ref[0, b]
    h_ref[0, 0] = jnp.dot(x, w, preferred_element_type=_f32)

    @pl.when(r == 0)
    def _():
        o_ref[0] = (jnp.dot(x, root_ref[0], preferred_element_type=_f32)
                    + bias_ref[0, 0])


def _dense2(o1, a1, basis, comp, root, bias):
    return pl.pallas_call(
        _dense2_body,
        grid=(2, R),
        in_specs=[
            pl.BlockSpec((1, NP, HID), lambda e, r: (e, 0, 0)),
            pl.BlockSpec((1, NP, HID), lambda e, r: (e, 0, 0)),
            pl.BlockSpec((1, R, HID, HID), lambda e, r: (e, 0, 0, 0)),
            pl.BlockSpec(memory_space=pltpu.SMEM),
            pl.BlockSpec((1, HID, HID), lambda e, r: (e, 0, 0)),
            pl.BlockSpec((1, 1, HID), lambda e, r: (e, 0, 0)),
        ],
        out_specs=[
            pl.BlockSpec((1, 1, NP, HID), lambda e, r: (e, r, 0, 0)),
            pl.BlockSpec((1, NP, HID), lambda e, r: (e, 0, 0)),
        ],
        out_shape=[
            jax.ShapeDtypeStruct((2, R, NP, HID), _f32),
            jax.ShapeDtypeStruct((2, NP, HID), _f32),
        ],
    )(o1, a1, basis, comp, root, bias)


def _head_body(o2_ref, a2_ref, batch_ref, depth_ref, demb_ref, wbt_ref,
               bbil_ref, w1a_ref, w1b_ref, w1c_ref, w1d_ref, w1e_ref,
               b1_ref, w2_ref, b2_ref, out_ref):
    pooled = []
    for e in range(2):
        x3 = jnp.maximum(o2_ref[e] + a2_ref[e], 0.0)
        bt = batch_ref[e, 0]
        oh = (lax.broadcasted_iota(_i32, (G, NP), 0) == bt[None, :])
        oh = oh.astype(_f32)
        ssum = jnp.dot(oh, x3, preferred_element_type=_f32)
        n = jnp.sum(oh, axis=1, keepdims=True)
        pooled.append(ssum / jnp.maximum(n, 1.0))
    hs, hg = pooled

    tt = jnp.dot(hs, wbt_ref[...], preferred_element_type=_f32)
    cols = []
    for k in range(CROSS):
        seg = tt[:, k * HID:(k + 1) * HID] * hg
        cols.append(jnp.sum(seg, axis=1, keepdims=True))
    cross = jnp.concatenate(cols, axis=1) + bbil_ref[...]

    dint = depth_ref[...]
    dfeat = dint.astype(_f32)
    dmin = jnp.minimum(dint, VOC - 1)
    ohd = (lax.broadcasted_iota(_i32, (G, VOC), 1) == dmin).astype(_f32)
    demb = jnp.dot(ohd, demb_ref[...], preferred_element_type=_f32)

    h1 = jnp.maximum(
        jnp.dot(hs, w1a_ref[...], preferred_element_type=_f32)
        + jnp.dot(hg, w1b_ref[...], preferred_element_type=_f32)
        + jnp.dot(cross, w1c_ref[...], preferred_element_type=_f32)
        + jnp.dot(dfeat, w1d_ref[...], preferred_element_type=_f32)
        + jnp.dot(demb, w1e_ref[...], preferred_element_type=_f32)
        + b1_ref[...], 0.0)
    out_ref[...] = jnp.dot(h1, w2_ref[...], preferred_element_type=_f32) \
        + b2_ref[...]


def _head(o2, a2, batch3, depth2, depth_emb, wbt, bbil, w1a, w1b, w1c, w1d,
          w1e, b1, w2, b2):
    return pl.pallas_call(
        _head_body,
        out_shape=jax.ShapeDtypeStruct((G, 1), _f32),
    )(o2, a2, batch3, depth2, depth_emb, wbt, bbil, w1a, w1b, w1c, w1d,
      w1e, b1, w2, b2)


def _pad1(a, n_to, val):
    return jnp.concatenate(
        [a, jnp.full((n_to - a.shape[0],), val, a.dtype)])


def kernel(x_s, edge_index_s, edge_type_s, batch_s, x_g, edge_index_g,
           edge_type_g, batch_g, depth, emb_s, comp1_s, basis1_s, root1_s,
           bias1_s, comp2_s, basis2_s, root2_s, bias2_s, emb_g, comp1_g,
           basis1_g, root1_g, bias1_g, comp2_g, basis2_g, root2_g, bias2_g,
           W_bil, b_bil, depth_emb, W1, b1, W2, b2):
    i32 = _i32
    ids = jnp.stack([_pad1(x_s.astype(i32), NP, 0),
                     _pad1(x_g.astype(i32), NP, 0)])
    src = jnp.stack([_pad1(edge_index_s[0].astype(i32), EP, 0),
                     _pad1(edge_index_g[0].astype(i32), EP, 0)])
    dst = jnp.stack([_pad1(edge_index_s[1].astype(i32), EP, N),
                     _pad1(edge_index_g[1].astype(i32), EP, N)])
    et = jnp.stack([_pad1(edge_type_s.astype(i32), EP, 0),
                    _pad1(edge_type_g.astype(i32), EP, 0)])
    embf = jnp.concatenate([emb_s, emb_g], axis=0)
    embf = jnp.pad(embf, ((0, 0), (0, EMBP - EMB)))

    x_sg, gk, sk, cntp = _pass_a_call()(ids, src, dst, et, embf)

    basis1 = jnp.pad(jnp.stack([basis1_s, basis1_g]),
                     ((0, 0), (0, 0), (0, EMBP - EMB), (0, 0)))
    comp1 = jnp.stack([comp1_s, comp1_g])
    root1 = jnp.pad(jnp.stack([root1_s, root1_g]),
                    ((0, 0), (0, EMBP - EMB), (0, 0)))
    bias1 = jnp.stack([bias1_s, bias1_g])[:, None, :]
    h1, o1, inv = _dense1(x_sg, basis1, comp1, root1, bias1,
                          cntp.reshape(2, NT, KR, 128))
    inv2 = inv.reshape(2, KT)
    scale = _pass_a2_call()(sk, inv2)
    meta = jnp.stack(
        [gk, jax.lax.bitcast_convert_type(scale, _i32)], axis=1)
    acc1 = _pass_b_call()(h1.reshape(2 * KT, HID), meta)

    basis2 = jnp.stack([basis2_s, basis2_g])
    comp2 = jnp.stack([comp2_s, comp2_g])
    root2 = jnp.stack([root2_s, root2_g])
    bias2 = jnp.stack([bias2_s, bias2_g])[:, None, :]
    h2, o2 = _dense2(o1, acc1, basis2, comp2, root2, bias2)
    acc2 = _pass_b_call()(h2.reshape(2 * KT, HID), meta)

    batch3 = jnp.stack([_pad1(batch_s.astype(i32), NP, G),
                        _pad1(batch_g.astype(i32), NP, G)])[:, None, :]
    depth2 = depth.astype(i32)[:, None]
    wbt = W_bil.transpose(1, 0, 2).reshape(HID, CROSS * HID)
    bbil = b_bil[None, :]
    w1a = W1[:HID]
    w1b = W1[HID:2 * HID]
    w1c = W1[2 * HID:2 * HID + CROSS]
    w1d = W1[2 * HID + CROSS:2 * HID + CROSS + 1]
    w1e = W1[2 * HID + CROSS + 1:]
    out = _head(o2, acc2, batch3, depth2, depth_emb, wbt, bbil,
                w1a, w1b, w1c, w1d, w1e, b1[None, :], W2, b2[None, :])
    return out[:, 0]


# speedup vs baseline: 17.8410x; 1.1871x over previous
import functools

import jax
import jax.numpy as jnp
from jax import lax
from jax.experimental import pallas as pl
from jax.experimental.pallas import tpu as pltpu
from jax.experimental.pallas import tpu_sc as plsc

N = 10000
NP = 10240
E = 320000
EP = 323584
R = 8
G = 16
EMB = 64
EMBP = 128
HID = 128
CROSS = 32
VOC = 512
DE = 8
NT = 16
PT = EP // NT
CT = PT // 128
KT = R * NP
KR = KT // 128

_f32 = jnp.float32
_i32 = jnp.int32


@functools.cache
def _mesh():
    return plsc.VectorSubcoreMesh(core_axis_name="c", subcore_axis_name="s",
                                  num_cores=2, num_subcores=NT)


def _pass_a(ids_h, src_h, dst_h, et_h, emb_h,
            x_out, gk_out, sk_out, cnt_out,
            idxb, xrows, srcb, dstb, etb, gkb, skb, cntb):
    cid = lax.axis_index("c")
    t = lax.axis_index("s")
    ones16 = jnp.ones((16,), _f32)
    zeros16 = jnp.zeros((16,), _f32)

    def _zrow(r, carry):
        cntb[pl.ds(r * 16, 16)] = zeros16
        return carry
    lax.fori_loop(0, KT // 16, _zrow, 0)

    def _xchunk(c, carry):
        row0 = pl.multiple_of(t * 640 + c * 128, 128)
        pltpu.sync_copy(ids_h.at[cid, pl.ds(row0, 128)], idxb)
        for j in range(8):
            s = pl.ds(j * 16, 16)
            idxb[s] = idxb[s] + cid * N
        pltpu.sync_copy(emb_h.at[idxb], xrows)
        pltpu.sync_copy(xrows, x_out.at[cid, pl.ds(row0, 128)])
        return carry
    lax.fori_loop(0, 5, _xchunk, 0)

    def _echunk(c, carry):
        base = pl.multiple_of(t * PT + c * 128, 128)
        pltpu.sync_copy(src_h.at[cid, pl.ds(base, 128)], srcb)
        pltpu.sync_copy(dst_h.at[cid, pl.ds(base, 128)], dstb)
        pltpu.sync_copy(et_h.at[cid, pl.ds(base, 128)], etb)
        for j in range(8):
            s = pl.ds(j * 16, 16)
            skey = dstb[s] * 8 + etb[s]
            plsc.addupdate_scatter(cntb, [skey], ones16)
            skb[s] = skey
            gkey = etb[s] * NP + srcb[s] + cid * KT
            gkb[s] = jnp.bitwise_or(gkey, lax.shift_left(dstb[s], 18))
        pltpu.sync_copy(gkb, gk_out.at[cid, pl.ds(base, 128)])
        pltpu.sync_copy(skb, sk_out.at[cid, pl.ds(base, 128)])
        return carry
    lax.fori_loop(0, CT, _echunk, 0)

    pltpu.sync_copy(cntb, cnt_out.at[cid, t])


@functools.cache
def _pass_a_call():
    return pl.kernel(
        _pass_a,
        out_type=[
            jax.ShapeDtypeStruct((2, NP, EMBP), _f32),
            jax.ShapeDtypeStruct((2, EP), _i32),
            jax.ShapeDtypeStruct((2, EP), _i32),
            jax.ShapeDtypeStruct((2, NT, KT), _f32),
        ],
        mesh=_mesh(),
        scratch_types=[
            pltpu.VMEM((128,), _i32),
            pltpu.VMEM((128, EMBP), _f32),
            pltpu.VMEM((128,), _i32),
            pltpu.VMEM((128,), _i32),
            pltpu.VMEM((128,), _i32),
            pltpu.VMEM((128,), _i32),
            pltpu.VMEM((128,), _i32),
            pltpu.VMEM((KT,), _f32),
        ],
        compiler_params=pltpu.CompilerParams(needs_layout_passes=False),
    )


def _pass_a2(sk_h, inv_h, sc_out, skb, scb, invb):
    cid = lax.axis_index("c")
    t = lax.axis_index("s")
    pltpu.sync_copy(inv_h.at[cid], invb)

    def _chunk(c, carry):
        base = pl.multiple_of(t * PT + c * 128, 128)
        pltpu.sync_copy(sk_h.at[cid, pl.ds(base, 128)], skb)
        for j in range(8):
            s = pl.ds(j * 16, 16)
            scb[s] = plsc.load_gather(invb, [skb[s]])
        pltpu.sync_copy(scb, sc_out.at[cid, pl.ds(base, 128)])
        return carry
    lax.fori_loop(0, CT, _chunk, 0)


@functools.cache
def _pass_a2_call():
    return pl.kernel(
        _pass_a2,
        out_type=jax.ShapeDtypeStruct((2, EP), _f32),
        mesh=_mesh(),
        scratch_types=[
            pltpu.VMEM((128,), _i32),
            pltpu.VMEM((128,), _f32),
            pltpu.VMEM((KT,), _f32),
        ],
        compiler_params=pltpu.CompilerParams(needs_layout_passes=False),
    )


def _pass_b(h_flat, meta_h, acc_out,
            rows, cb, sb, gkb, dstb, scb, accs, gsem0, gsem1, ssem0, ssem1):
    cid = lax.axis_index("c")
    t = lax.axis_index("s")
    zeros16 = jnp.zeros((16,), _f32)
    gsems = (gsem0, gsem1)
    ssems = (ssem0, ssem1)

    def _zrow(r, carry):
        for j in range(8):
            rows[0, r, pl.ds(j * 16, 16)] = zeros16
        return carry
    lax.fori_loop(0, 128, _zrow, 0)
    for c in range(5):
        pltpu.sync_copy(rows.at[0], accs.at[pl.ds(t * 640 + c * 128, 128)])
    plsc.subcore_barrier()

    def _fetch(c, slot):
        base = pl.multiple_of(t * PT + c * 128, 128)
        pltpu.sync_copy(meta_h.at[cid, 0, pl.ds(base, 128)], cb.at[slot])
        pltpu.sync_copy(meta_h.at[cid, 1, pl.ds(base, 128)], sb.at[slot])
        for j in range(8):
            s = pl.ds(j * 16, 16)
            cv = cb[slot, s]
            gkb[slot, s] = jnp.bitwise_and(cv, (1 << 18) - 1)
            dstb[slot, s] = lax.shift_right_logical(cv, 18)
            scb[slot, s] = plsc.bitcast(sb[slot, s], _f32)
        pltpu.async_copy(h_flat.at[gkb.at[slot]], rows.at[slot], gsems[slot])

    _fetch(0, 0)
    _fetch(1, 1)

    def _pair(p, carry):
        for slot in range(2):
            c = 2 * p + slot
            pltpu.make_async_copy(
                h_flat.at[gkb.at[slot]], rows.at[slot], gsems[slot]).wait()

            def _edge(e, c2):
                sp = plsc.load_gather(scb.at[slot],
                                      [jnp.full((16,), e, _i32)])
                for j in range(8):
                    s = pl.ds(j * 16, 16)
                    rows[slot, e, s] = rows[slot, e, s] * sp
                return c2
            lax.fori_loop(0, 128, _edge, 0)
            pltpu.async_copy(rows.at[slot], accs.at[dstb.at[slot]],
                             ssems[slot], add=True)

            @pl.when(c + 2 < CT)
            def _():
                pltpu.make_async_copy(
                    rows.at[slot], accs.at[dstb.at[slot]],
                    ssems[slot]).wait()
                _fetch(c + 2, slot)
        return carry
    lax.fori_loop(0, CT // 2, _pair, 0)

    for slot in range(2):
        pltpu.make_async_copy(
            rows.at[slot], accs.at[dstb.at[slot]], ssems[slot]).wait()
    plsc.subcore_barrier()

    for c in range(5):
        r0 = t * 640 + c * 128
        pltpu.sync_copy(accs.at[pl.ds(r0, 128)],
                        acc_out.at[cid, pl.ds(r0, 128)])


@functools.cache
def _pass_b_call():
    return pl.kernel(
        _pass_b,
        out_type=jax.ShapeDtypeStruct((2, NP, HID), _f32),
        mesh=_mesh(),
        scratch_types=[
            pltpu.VMEM((2, 128, HID), _f32),
            pltpu.VMEM((2, 128), _i32),
            pltpu.VMEM((2, 128), _i32),
            pltpu.VMEM((2, 128), _i32),
            pltpu.VMEM((2, 128), _i32),
            pltpu.VMEM((2, 128), _f32),
            pltpu.VMEM_SHARED((NP, HID), _f32),
            pltpu.SemaphoreType.DMA,
            pltpu.SemaphoreType.DMA,
            pltpu.SemaphoreType.DMA,
            pltpu.SemaphoreType.DMA,
        ],
        compiler_params=pltpu.CompilerParams(needs_layout_passes=False),
    )


def _dense1_body(x_ref, basis_ref, comp_ref, root_ref, bias_ref, cnt_ref,
                 h_ref, o_ref, inv_ref):
    e = pl.program_id(0)
    r = pl.program_id(1)
    x = x_ref[0]
    w = comp_ref[e, r, 0] * basis_ref[0, 0]
    for b in range(1, R):
        w = w + comp_ref[e, r, b] * basis_ref[0, b]
    h_ref[0, 0] = jnp.dot(x, w, preferred_element_type=_f32)

    @pl.when(r == 0)
    def _():
        o_ref[0] = (jnp.dot(x, root_ref[0], preferred_element_type=_f32)
                    + bias_ref[0, 0])
        cnt = jnp.sum(cnt_ref[0], axis=0)
        inv_ref[0] = 1.0 / jnp.maximum(cnt, 1.0)


def _dense1(x, basis, comp, root, bias, cntp):
    return pl.pallas_call(
        _dense1_body,
        grid=(2, R),
        in_specs=[
            pl.BlockSpec((1, NP, EMBP), lambda e, r: (e, 0, 0)),
            pl.BlockSpec((1, R, EMBP, HID), lambda e, r: (e, 0, 0, 0)),
            pl.BlockSpec(memory_space=pltpu.SMEM),
            pl.BlockSpec((1, EMBP, HID), lambda e, r: (e, 0, 0)),
            pl.BlockSpec((1, 1, HID), lambda e, r: (e, 0, 0)),
            pl.BlockSpec((1, NT, KR, 128), lambda e, r: (e, 0, 0, 0)),
        ],
        out_specs=[
            pl.BlockSpec((1, 1, NP, HID), lambda e, r: (e, r, 0, 0)),
            pl.BlockSpec((1, NP, HID), lambda e, r: (e, 0, 0)),
            pl.BlockSpec((1, KR, 128), lambda e, r: (e, 0, 0)),
        ],
        out_shape=[
            jax.ShapeDtypeStruct((2, R, NP, HID), _f32),
            jax.ShapeDtypeStruct((2, NP, HID), _f32),
            jax.ShapeDtypeStruct((2, KR, 128), _f32),
        ],
    )(x, basis, comp, root, bias, cntp)


def _dense2_body(o1_ref, a1_ref, basis_ref, comp_ref, root_ref, bias_ref,
                 h_ref, o_ref):
    e = pl.program_id(0)
    r = pl.program_id(1)
    x = jnp.maximum(o1_ref[0] + a1_ref[0], 0.0)
    w = comp_ref[e, r, 0] * basis_ref[0, 0]
    for b in range(1, R):
        w = w + comp_ref[e, r, b] * basis_ref[0, b]
    h_ref[0, 0] = jnp.dot(x, w, preferred_element_type=_f32)

    @pl.when(r == 0)
    def _():
        o_ref[0] = (jnp.dot(x, root_ref[0], preferred_element_type=_f32)
                    + bias_ref[0, 0])


def _dense2(o1, a1, basis, comp, root, bias):
    return pl.pallas_call(
        _dense2_body,
        grid=(2, R),
        in_specs=[
            pl.BlockSpec((1, NP, HID), lambda e, r: (e, 0, 0)),
            pl.BlockSpec((1, NP, HID), lambda e, r: (e, 0, 0)),
            pl.BlockSpec((1, R, HID, HID), lambda e, r: (e, 0, 0, 0)),
            pl.BlockSpec(memory_space=pltpu.SMEM),
            pl.BlockSpec((1, HID, HID), lambda e, r: (e, 0, 0)),
            pl.BlockSpec((1, 1, HID), lambda e, r: (e, 0, 0)),
        ],
        out_specs=[
            pl.BlockSpec((1, 1, NP, HID), lambda e, r: (e, r, 0, 0)),
            pl.BlockSpec((1, NP, HID), lambda e, r: (e, 0, 0)),
        ],
        out_shape=[
            jax.ShapeDtypeStruct((2, R, NP, HID), _f32),
            jax.ShapeDtypeStruct((2, NP, HID), _f32),
        ],
    )(o1, a1, basis, comp, root, bias)


def _head_body(o2_ref, a2_ref, batch_ref, depth_ref, demb_ref, wbt_ref,
               bbil_ref, w1a_ref, w1b_ref, w1c_ref, w1d_ref, w1e_ref,
               b1_ref, w2_ref, b2_ref, out_ref):
    pooled = []
    for e in range(2):
        x3 = jnp.maximum(o2_ref[e] + a2_ref[e], 0.0)
        bt = batch_ref[e, 0]
        oh = (lax.broadcasted_iota(_i32, (G, NP), 0) == bt[None, :])
        oh = oh.astype(_f32)
        ssum = jnp.dot(oh, x3, preferred_element_type=_f32)
        n = jnp.sum(oh, axis=1, keepdims=True)
        pooled.append(ssum / jnp.maximum(n, 1.0))
    hs, hg = pooled

    tt = jnp.dot(hs, wbt_ref[...], preferred_element_type=_f32)
    cols = []
    for k in range(CROSS):
        seg = tt[:, k * HID:(k + 1) * HID] * hg
        cols.append(jnp.sum(seg, axis=1, keepdims=True))
    cross = jnp.concatenate(cols, axis=1) + bbil_ref[...]

    dint = depth_ref[...]
    dfeat = dint.astype(_f32)
    dmin = jnp.minimum(dint, VOC - 1)
    ohd = (lax.broadcasted_iota(_i32, (G, VOC), 1) == dmin).astype(_f32)
    demb = jnp.dot(ohd, demb_ref[...], preferred_element_type=_f32)

    h1 = jnp.maximum(
        jnp.dot(hs, w1a_ref[...], preferred_element_type=_f32)
        + jnp.dot(hg, w1b_ref[...], preferred_element_type=_f32)
        + jnp.dot(cross, w1c_ref[...], preferred_element_type=_f32)
        + jnp.dot(dfeat, w1d_ref[...], preferred_element_type=_f32)
        + jnp.dot(demb, w1e_ref[...], preferred_element_type=_f32)
        + b1_ref[...], 0.0)
    out_ref[...] = jnp.dot(h1, w2_ref[...], preferred_element_type=_f32) \
        + b2_ref[...]


def _head(o2, a2, batch3, depth2, depth_emb, wbt, bbil, w1a, w1b, w1c, w1d,
          w1e, b1, w2, b2):
    return pl.pallas_call(
        _head_body,
        out_shape=jax.ShapeDtypeStruct((G, 1), _f32),
    )(o2, a2, batch3, depth2, depth_emb, wbt, bbil, w1a, w1b, w1c, w1d,
      w1e, b1, w2, b2)


def _pad1(a, n_to, val):
    return jnp.concatenate(
        [a, jnp.full((n_to - a.shape[0],), val, a.dtype)])


def kernel(x_s, edge_index_s, edge_type_s, batch_s, x_g, edge_index_g,
           edge_type_g, batch_g, depth, emb_s, comp1_s, basis1_s, root1_s,
           bias1_s, comp2_s, basis2_s, root2_s, bias2_s, emb_g, comp1_g,
           basis1_g, root1_g, bias1_g, comp2_g, basis2_g, root2_g, bias2_g,
           W_bil, b_bil, depth_emb, W1, b1, W2, b2):
    i32 = _i32
    ids = jnp.stack([_pad1(x_s.astype(i32), NP, 0),
                     _pad1(x_g.astype(i32), NP, 0)])
    src = jnp.stack([_pad1(edge_index_s[0].astype(i32), EP, 0),
                     _pad1(edge_index_g[0].astype(i32), EP, 0)])
    dst = jnp.stack([_pad1(edge_index_s[1].astype(i32), EP, N),
                     _pad1(edge_index_g[1].astype(i32), EP, N)])
    et = jnp.stack([_pad1(edge_type_s.astype(i32), EP, 0),
                    _pad1(edge_type_g.astype(i32), EP, 0)])
    embf = jnp.concatenate([emb_s, emb_g], axis=0)
    embf = jnp.pad(embf, ((0, 0), (0, EMBP - EMB)))

    x_sg, gk, sk, cntp = _pass_a_call()(ids, src, dst, et, embf)

    basis1 = jnp.pad(jnp.stack([basis1_s, basis1_g]),
                     ((0, 0), (0, 0), (0, EMBP - EMB), (0, 0)))
    comp1 = jnp.stack([comp1_s, comp1_g])
    root1 = jnp.pad(jnp.stack([root1_s, root1_g]),
                    ((0, 0), (0, EMBP - EMB), (0, 0)))
    bias1 = jnp.stack([bias1_s, bias1_g])[:, None, :]
    h1, o1, inv = _dense1(x_sg, basis1, comp1, root1, bias1,
                          cntp.reshape(2, NT, KR, 128))
    inv2 = inv.reshape(2, KT)
    scale = _pass_a2_call()(sk, inv2)
    meta = jnp.stack(
        [gk, jax.lax.bitcast_convert_type(scale, _i32)], axis=1)
    acc1 = _pass_b_call()(h1.reshape(2 * KT, HID), meta)

    basis2 = jnp.stack([basis2_s, basis2_g])
    comp2 = jnp.stack([comp2_s, comp2_g])
    root2 = jnp.stack([root2_s, root2_g])
    bias2 = jnp.stack([bias2_s, bias2_g])[:, None, :]
    h2, o2 = _dense2(o1, acc1, basis2, comp2, root2, bias2)
    acc2 = _pass_b_call()(h2.reshape(2 * KT, HID), meta)

    batch3 = jnp.stack([_pad1(batch_s.astype(i32), NP, G),
                        _pad1(batch_g.astype(i32), NP, G)])[:, None, :]
    depth2 = depth.astype(i32)[:, None]
    wbt = W_bil.transpose(1, 0, 2).reshape(HID, CROSS * HID)
    bbil = b_bil[None, :]
    w1a = W1[:HID]
    w1b = W1[HID:2 * HID]
    w1c = W1[2 * HID:2 * HID + CROSS]
    w1d = W1[2 * HID + CROSS:2 * HID + CROSS + 1]
    w1e = W1[2 * HID + CROSS + 1:]
    out = _head(o2, acc2, batch3, depth2, depth_emb, wbt, bbil,
                w1a, w1b, w1c, w1d, w1e, b1[None, :], W2, b2[None, :])
    return out[:, 0]


# speedup vs baseline: 19.9685x; 1.1192x over previous
import functools

import jax
import jax.numpy as jnp
from jax import lax
from jax.experimental import pallas as pl
from jax.experimental.pallas import tpu as pltpu
from jax.experimental.pallas import tpu_sc as plsc

N = 10000
NP = 10240
E = 320000
EP = 323584
R = 8
G = 16
EMB = 64
EMBP = 128
HID = 128
CROSS = 32
VOC = 512
DE = 8
NT = 16
PT = EP // NT
CH = 128
CT = PT // CH
KT = R * NP
KR = KT // 128

_f32 = jnp.float32
_i32 = jnp.int32


@functools.cache
def _mesh():
    return plsc.VectorSubcoreMesh(core_axis_name="c", subcore_axis_name="s",
                                  num_cores=2, num_subcores=NT)


def _pass_a(ids_h, src_h, dst_h, et_h, emb_h,
            x_out, gk_out, sk_out, cnt_out,
            idxb, xrows, srcb, dstb, etb, gkb, skb, cntb):
    cid = lax.axis_index("c")
    t = lax.axis_index("s")
    ones16 = jnp.ones((16,), _f32)
    zeros16 = jnp.zeros((16,), _f32)

    def _zrow(r, carry):
        cntb[pl.ds(r * 16, 16)] = zeros16
        return carry
    lax.fori_loop(0, KT // 16, _zrow, 0)

    def _xchunk(c, carry):
        row0 = pl.multiple_of(t * 640 + c * 128, 128)
        pltpu.sync_copy(ids_h.at[cid, pl.ds(row0, 128)], idxb)
        for j in range(8):
            s = pl.ds(j * 16, 16)
            idxb[s] = idxb[s] + cid * N
        pltpu.sync_copy(emb_h.at[idxb], xrows)
        pltpu.sync_copy(xrows, x_out.at[cid, pl.ds(row0, 128)])
        return carry
    lax.fori_loop(0, 5, _xchunk, 0)

    def _echunk(c, carry):
        base = pl.multiple_of(t * PT + c * CH, 16)
        pltpu.sync_copy(src_h.at[cid, pl.ds(base, CH)], srcb)
        pltpu.sync_copy(dst_h.at[cid, pl.ds(base, CH)], dstb)
        pltpu.sync_copy(et_h.at[cid, pl.ds(base, CH)], etb)
        for j in range(CH // 16):
            s = pl.ds(j * 16, 16)
            skey = dstb[s] * 8 + etb[s]
            plsc.addupdate_scatter(cntb, [skey], ones16)
            skb[s] = skey
            gkey = etb[s] * NP + srcb[s] + cid * KT
            gkb[s] = jnp.bitwise_or(gkey, lax.shift_left(dstb[s], 18))
        pltpu.sync_copy(gkb, gk_out.at[cid, pl.ds(base, CH)])
        pltpu.sync_copy(skb, sk_out.at[cid, pl.ds(base, CH)])
        return carry
    lax.fori_loop(0, CT, _echunk, 0)

    pltpu.sync_copy(cntb, cnt_out.at[cid, t])


@functools.cache
def _pass_a_call():
    return pl.kernel(
        _pass_a,
        out_type=[
            jax.ShapeDtypeStruct((2, NP, EMBP), _f32),
            jax.ShapeDtypeStruct((2, EP), _i32),
            jax.ShapeDtypeStruct((2, EP), _i32),
            jax.ShapeDtypeStruct((2, NT, KT), _f32),
        ],
        mesh=_mesh(),
        scratch_types=[
            pltpu.VMEM((128,), _i32),
            pltpu.VMEM((128, EMBP), _f32),
            pltpu.VMEM((CH,), _i32),
            pltpu.VMEM((CH,), _i32),
            pltpu.VMEM((CH,), _i32),
            pltpu.VMEM((CH,), _i32),
            pltpu.VMEM((CH,), _i32),
            pltpu.VMEM((KT,), _f32),
        ],
        compiler_params=pltpu.CompilerParams(needs_layout_passes=False),
    )


def _pass_a2(sk_h, inv_h, sc_out, skb, scb, invb):
    cid = lax.axis_index("c")
    t = lax.axis_index("s")
    pltpu.sync_copy(inv_h.at[cid], invb)

    def _chunk(c, carry):
        base = pl.multiple_of(t * PT + c * CH, 16)
        pltpu.sync_copy(sk_h.at[cid, pl.ds(base, CH)], skb)
        for j in range(CH // 16):
            s = pl.ds(j * 16, 16)
            scb[s] = plsc.load_gather(invb, [skb[s]])
        pltpu.sync_copy(scb, sc_out.at[cid, pl.ds(base, CH)])
        return carry
    lax.fori_loop(0, CT, _chunk, 0)


@functools.cache
def _pass_a2_call():
    return pl.kernel(
        _pass_a2,
        out_type=jax.ShapeDtypeStruct((2, EP), _f32),
        mesh=_mesh(),
        scratch_types=[
            pltpu.VMEM((CH,), _i32),
            pltpu.VMEM((CH,), _f32),
            pltpu.VMEM((KT,), _f32),
        ],
        compiler_params=pltpu.CompilerParams(needs_layout_passes=False),
    )


def _pass_b(h_flat, meta_h, acc_out,
            rows, cb, sb, gkb, dstb, scb, accs,
            gsem0, gsem1, ssem0, ssem1):
    cid = lax.axis_index("c")
    t = lax.axis_index("s")
    zeros16 = jnp.zeros((16,), _f32)
    gsems = (gsem0, gsem1)
    ssems = (ssem0, ssem1)

    def _zrow(r, carry):
        for j in range(8):
            rows[0, r, pl.ds(j * 16, 16)] = zeros16
        return carry
    lax.fori_loop(0, CH, _zrow, 0)
    for c in range(5):
        pltpu.sync_copy(rows.at[0], accs.at[pl.ds(t * 640 + c * CH, CH)])
    plsc.subcore_barrier()

    def _load_meta(c, slot):
        base = pl.multiple_of(t * PT + c * CH, CH)
        pltpu.sync_copy(meta_h.at[cid, 0, pl.ds(base, CH)], cb.at[slot])
        pltpu.sync_copy(meta_h.at[cid, 1, pl.ds(base, CH)], sb.at[slot])

    def _launch(slot):
        for j in range(CH // 16):
            s = pl.ds(j * 16, 16)
            cv = cb[slot, s]
            gkb[slot, s] = jnp.bitwise_and(cv, (1 << 18) - 1)
            dstb[slot, s] = lax.shift_right_logical(cv, 18)
            scb[slot, s] = plsc.bitcast(sb[slot, s], _f32)
        pltpu.async_copy(h_flat.at[gkb.at[slot]], rows.at[slot], gsems[slot])

    def _wait_scatter(slot):
        pltpu.make_async_copy(
            rows.at[slot], accs.at[dstb.at[slot]], ssems[slot]).wait()

    def _process(slot):
        pltpu.make_async_copy(
            h_flat.at[gkb.at[slot]], rows.at[slot], gsems[slot]).wait()

        def _edge(e, c2):
            for u in range(2):
                ee = e * 2 + u
                sp = plsc.load_gather(scb.at[slot],
                                      [jnp.full((16,), ee, _i32)])
                for j in range(8):
                    s = pl.ds(j * 16, 16)
                    rows[slot, ee, s] = rows[slot, ee, s] * sp
            return c2
        lax.fori_loop(0, CH // 2, _edge, 0)
        pltpu.async_copy(rows.at[slot], accs.at[dstb.at[slot]],
                         ssems[slot], add=True)

    _load_meta(0, 0)
    _launch(0)
    _load_meta(1, 1)
    _launch(1)
    _process(0)

    def _group(p, carry):
        for s in range(2):
            i = 2 * p + s
            _load_meta(i, s)
            _wait_scatter(s)
            _launch(s)
            _process(1 - s)
        return carry
    lax.fori_loop(1, CT // 2, _group, 0)

    _process(1)
    for slot in range(2):
        _wait_scatter(slot)
    plsc.subcore_barrier()

    pltpu.sync_copy(accs.at[pl.ds(t * 640, 640)],
                    acc_out.at[cid, pl.ds(t * 640, 640)])


@functools.cache
def _pass_b_call():
    return pl.kernel(
        _pass_b,
        out_type=jax.ShapeDtypeStruct((2, NP, HID), _f32),
        mesh=_mesh(),
        scratch_types=[
            pltpu.VMEM((2, CH, HID), _f32),
            pltpu.VMEM((2, CH), _i32),
            pltpu.VMEM((2, CH), _i32),
            pltpu.VMEM((2, CH), _i32),
            pltpu.VMEM((2, CH), _i32),
            pltpu.VMEM((2, CH), _f32),
            pltpu.VMEM_SHARED((NP, HID), _f32),
            pltpu.SemaphoreType.DMA,
            pltpu.SemaphoreType.DMA,
            pltpu.SemaphoreType.DMA,
            pltpu.SemaphoreType.DMA,
        ],
        compiler_params=pltpu.CompilerParams(needs_layout_passes=False),
    )


def _dense1_body(x_ref, basis_ref, comp_ref, root_ref, bias_ref, cnt_ref,
                 h_ref, o_ref, inv_ref):
    e = pl.program_id(0)
    r = pl.program_id(1)
    x = x_ref[0]
    w = comp_ref[e, r, 0] * basis_ref[0, 0]
    for b in range(1, R):
        w = w + comp_ref[e, r, b] * basis_ref[0, b]
    h_ref[0, 0] = jnp.dot(x, w, preferred_element_type=_f32)

    @pl.when(r == 0)
    def _():
        o_ref[0] = (jnp.dot(x, root_ref[0], preferred_element_type=_f32)
                    + bias_ref[0, 0])
        cnt = jnp.sum(cnt_ref[0], axis=0)
        inv_ref[0] = 1.0 / jnp.maximum(cnt, 1.0)


def _dense1(x, basis, comp, root, bias, cntp):
    return pl.pallas_call(
        _dense1_body,
        grid=(2, R),
        in_specs=[
            pl.BlockSpec((1, NP, EMBP), lambda e, r: (e, 0, 0)),
            pl.BlockSpec((1, R, EMBP, HID), lambda e, r: (e, 0, 0, 0)),
            pl.BlockSpec(memory_space=pltpu.SMEM),
            pl.BlockSpec((1, EMBP, HID), lambda e, r: (e, 0, 0)),
            pl.BlockSpec((1, 1, HID), lambda e, r: (e, 0, 0)),
            pl.BlockSpec((1, NT, KR, 128), lambda e, r: (e, 0, 0, 0)),
        ],
        out_specs=[
            pl.BlockSpec((1, 1, NP, HID), lambda e, r: (e, r, 0, 0)),
            pl.BlockSpec((1, NP, HID), lambda e, r: (e, 0, 0)),
            pl.BlockSpec((1, KR, 128), lambda e, r: (e, 0, 0)),
        ],
        out_shape=[
            jax.ShapeDtypeStruct((2, R, NP, HID), _f32),
            jax.ShapeDtypeStruct((2, NP, HID), _f32),
            jax.ShapeDtypeStruct((2, KR, 128), _f32),
        ],
    )(x, basis, comp, root, bias, cntp)


def _dense2_body(o1_ref, a1_ref, basis_ref, comp_ref, root_ref, bias_ref,
                 h_ref, o_ref):
    e = pl.program_id(0)
    r = pl.program_id(1)
    x = jnp.maximum(o1_ref[0] + a1_ref[0], 0.0)
    w = comp_ref[e, r, 0] * basis_ref[0, 0]
    for b in range(1, R):
        w = w + comp_ref[e, r, b] * basis_ref[0, b]
    h_ref[0, 0] = jnp.dot(x, w, preferred_element_type=_f32)

    @pl.when(r == 0)
    def _():
        o_ref[0] = (jnp.dot(x, root_ref[0], preferred_element_type=_f32)
                    + bias_ref[0, 0])


def _dense2(o1, a1, basis, comp, root, bias):
    return pl.pallas_call(
        _dense2_body,
        grid=(2, R),
        in_specs=[
            pl.BlockSpec((1, NP, HID), lambda e, r: (e, 0, 0)),
            pl.BlockSpec((1, NP, HID), lambda e, r: (e, 0, 0)),
            pl.BlockSpec((1, R, HID, HID), lambda e, r: (e, 0, 0, 0)),
            pl.BlockSpec(memory_space=pltpu.SMEM),
            pl.BlockSpec((1, HID, HID), lambda e, r: (e, 0, 0)),
            pl.BlockSpec((1, 1, HID), lambda e, r: (e, 0, 0)),
        ],
        out_specs=[
            pl.BlockSpec((1, 1, NP, HID), lambda e, r: (e, r, 0, 0)),
            pl.BlockSpec((1, NP, HID), lambda e, r: (e, 0, 0)),
        ],
        out_shape=[
            jax.ShapeDtypeStruct((2, R, NP, HID), _f32),
            jax.ShapeDtypeStruct((2, NP, HID), _f32),
        ],
    )(o1, a1, basis, comp, root, bias)


def _head_body(o2_ref, a2_ref, batch_ref, depth_ref, demb_ref, wbt_ref,
               bbil_ref, w1a_ref, w1b_ref, w1c_ref, w1d_ref, w1e_ref,
               b1_ref, w2_ref, b2_ref, out_ref):
    pooled = []
    for e in range(2):
        x3 = jnp.maximum(o2_ref[e] + a2_ref[e], 0.0)
        bt = batch_ref[e, 0]
        oh = (lax.broadcasted_iota(_i32, (G, NP), 0) == bt[None, :])
        oh = oh.astype(_f32)
        ssum = jnp.dot(oh, x3, preferred_element_type=_f32)
        n = jnp.sum(oh, axis=1, keepdims=True)
        pooled.append(ssum / jnp.maximum(n, 1.0))
    hs, hg = pooled

    tt = jnp.dot(hs, wbt_ref[...], preferred_element_type=_f32)
    cols = []
    for k in range(CROSS):
        seg = tt[:, k * HID:(k + 1) * HID] * hg
        cols.append(jnp.sum(seg, axis=1, keepdims=True))
    cross = jnp.concatenate(cols, axis=1) + bbil_ref[...]

    dint = depth_ref[...]
    dfeat = dint.astype(_f32)
    dmin = jnp.minimum(dint, VOC - 1)
    ohd = (lax.broadcasted_iota(_i32, (G, VOC), 1) == dmin).astype(_f32)
    demb = jnp.dot(ohd, demb_ref[...], preferred_element_type=_f32)

    h1 = jnp.maximum(
        jnp.dot(hs, w1a_ref[...], preferred_element_type=_f32)
        + jnp.dot(hg, w1b_ref[...], preferred_element_type=_f32)
        + jnp.dot(cross, w1c_ref[...], preferred_element_type=_f32)
        + jnp.dot(dfeat, w1d_ref[...], preferred_element_type=_f32)
        + jnp.dot(demb, w1e_ref[...], preferred_element_type=_f32)
        + b1_ref[...], 0.0)
    out_ref[...] = jnp.dot(h1, w2_ref[...], preferred_element_type=_f32) \
        + b2_ref[...]


def _head(o2, a2, batch3, depth2, depth_emb, wbt, bbil, w1a, w1b, w1c, w1d,
          w1e, b1, w2, b2):
    return pl.pallas_call(
        _head_body,
        out_shape=jax.ShapeDtypeStruct((G, 1), _f32),
    )(o2, a2, batch3, depth2, depth_emb, wbt, bbil, w1a, w1b, w1c, w1d,
      w1e, b1, w2, b2)


def _pad1(a, n_to, val):
    return jnp.concatenate(
        [a, jnp.full((n_to - a.shape[0],), val, a.dtype)])


def kernel(x_s, edge_index_s, edge_type_s, batch_s, x_g, edge_index_g,
           edge_type_g, batch_g, depth, emb_s, comp1_s, basis1_s, root1_s,
           bias1_s, comp2_s, basis2_s, root2_s, bias2_s, emb_g, comp1_g,
           basis1_g, root1_g, bias1_g, comp2_g, basis2_g, root2_g, bias2_g,
           W_bil, b_bil, depth_emb, W1, b1, W2, b2):
    i32 = _i32
    ids = jnp.stack([_pad1(x_s.astype(i32), NP, 0),
                     _pad1(x_g.astype(i32), NP, 0)])
    src = jnp.stack([_pad1(edge_index_s[0].astype(i32), EP, 0),
                     _pad1(edge_index_g[0].astype(i32), EP, 0)])
    dst = jnp.stack([_pad1(edge_index_s[1].astype(i32), EP, N),
                     _pad1(edge_index_g[1].astype(i32), EP, N)])
    et = jnp.stack([_pad1(edge_type_s.astype(i32), EP, 0),
                    _pad1(edge_type_g.astype(i32), EP, 0)])
    embf = jnp.concatenate([emb_s, emb_g], axis=0)
    embf = jnp.pad(embf, ((0, 0), (0, EMBP - EMB)))

    x_sg, gk, sk, cntp = _pass_a_call()(ids, src, dst, et, embf)

    basis1 = jnp.pad(jnp.stack([basis1_s, basis1_g]),
                     ((0, 0), (0, 0), (0, EMBP - EMB), (0, 0)))
    comp1 = jnp.stack([comp1_s, comp1_g])
    root1 = jnp.pad(jnp.stack([root1_s, root1_g]),
                    ((0, 0), (0, EMBP - EMB), (0, 0)))
    bias1 = jnp.stack([bias1_s, bias1_g])[:, None, :]
    h1, o1, inv = _dense1(x_sg, basis1, comp1, root1, bias1,
                          cntp.reshape(2, NT, KR, 128))
    inv2 = inv.reshape(2, KT)
    scale = _pass_a2_call()(sk, inv2)
    meta = jnp.stack(
        [gk, jax.lax.bitcast_convert_type(scale, _i32)], axis=1)
    acc1 = _pass_b_call()(h1.reshape(2 * KT, HID), meta)

    basis2 = jnp.stack([basis2_s, basis2_g])
    comp2 = jnp.stack([comp2_s, comp2_g])
    root2 = jnp.stack([root2_s, root2_g])
    bias2 = jnp.stack([bias2_s, bias2_g])[:, None, :]
    h2, o2 = _dense2(o1, acc1, basis2, comp2, root2, bias2)
    acc2 = _pass_b_call()(h2.reshape(2 * KT, HID), meta)

    batch3 = jnp.stack([_pad1(batch_s.astype(i32), NP, G),
                        _pad1(batch_g.astype(i32), NP, G)])[:, None, :]
    depth2 = depth.astype(i32)[:, None]
    wbt = W_bil.transpose(1, 0, 2).reshape(HID, CROSS * HID)
    bbil = b_bil[None, :]
    w1a = W1[:HID]
    w1b = W1[HID:2 * HID]
    w1c = W1[2 * HID:2 * HID + CROSS]
    w1d = W1[2 * HID + CROSS:2 * HID + CROSS + 1]
    w1e = W1[2 * HID + CROSS + 1:]
    out = _head(o2, acc2, batch3, depth2, depth_emb, wbt, bbil,
                w1a, w1b, w1c, w1d, w1e, b1[None, :], W2, b2[None, :])
    return out[:, 0]


# speedup vs baseline: 19.9849x; 1.0008x over previous
import functools

import jax
import jax.numpy as jnp
from jax import lax
from jax.experimental import pallas as pl
from jax.experimental.pallas import tpu as pltpu
from jax.experimental.pallas import tpu_sc as plsc

N = 10000
NP = 10240
E = 320000
EP = 323584
R = 8
G = 16
EMB = 64
EMBP = 128
HID = 128
CROSS = 32
VOC = 512
DE = 8
NT = 16
PT = EP // NT
CH = 128
CT = PT // CH
KT = R * NP
KR = KT // 128

_f32 = jnp.float32
_i32 = jnp.int32


@functools.cache
def _mesh():
    return plsc.VectorSubcoreMesh(core_axis_name="c", subcore_axis_name="s",
                                  num_cores=2, num_subcores=NT)


def _pass_a(ids_h, src_h, dst_h, et_h, emb_h,
            x_out, gk_out, sk_out, cnt_out,
            idxb, xrows, srcb, dstb, etb, gkb, skb, cntb):
    cid = lax.axis_index("c")
    t = lax.axis_index("s")
    ones16 = jnp.ones((16,), _f32)
    zeros16 = jnp.zeros((16,), _f32)

    def _zrow(r, carry):
        cntb[pl.ds(r * 16, 16)] = zeros16
        return carry
    lax.fori_loop(0, KT // 16, _zrow, 0)

    def _xchunk(c, carry):
        row0 = pl.multiple_of(t * 640 + c * 128, 128)
        pltpu.sync_copy(ids_h.at[cid, pl.ds(row0, 128)], idxb)
        for j in range(8):
            s = pl.ds(j * 16, 16)
            idxb[s] = idxb[s] + cid * N
        pltpu.sync_copy(emb_h.at[idxb], xrows)
        pltpu.sync_copy(xrows, x_out.at[cid, pl.ds(row0, 128)])
        return carry
    lax.fori_loop(0, 5, _xchunk, 0)

    def _echunk(c, carry):
        base = pl.multiple_of(t * PT + c * CH, 16)
        pltpu.sync_copy(src_h.at[cid, pl.ds(base, CH)], srcb)
        pltpu.sync_copy(dst_h.at[cid, pl.ds(base, CH)], dstb)
        pltpu.sync_copy(et_h.at[cid, pl.ds(base, CH)], etb)
        for j in range(CH // 16):
            s = pl.ds(j * 16, 16)
            skey = dstb[s] * 8 + etb[s]
            plsc.addupdate_scatter(cntb, [skey], ones16)
            skb[s] = skey
            gkey = etb[s] * NP + srcb[s] + cid * KT
            gkb[s] = jnp.bitwise_or(gkey, lax.shift_left(dstb[s], 18))
        pltpu.sync_copy(gkb, gk_out.at[cid, pl.ds(base, CH)])
        pltpu.sync_copy(skb, sk_out.at[cid, pl.ds(base, CH)])
        return carry
    lax.fori_loop(0, CT, _echunk, 0)

    pltpu.sync_copy(cntb, cnt_out.at[cid, t])


@functools.cache
def _pass_a_call():
    return pl.kernel(
        _pass_a,
        out_type=[
            jax.ShapeDtypeStruct((2, NP, EMBP), _f32),
            jax.ShapeDtypeStruct((2, EP), _i32),
            jax.ShapeDtypeStruct((2, EP), _i32),
            jax.ShapeDtypeStruct((2, NT, KT), _f32),
        ],
        mesh=_mesh(),
        scratch_types=[
            pltpu.VMEM((128,), _i32),
            pltpu.VMEM((128, EMBP), _f32),
            pltpu.VMEM((CH,), _i32),
            pltpu.VMEM((CH,), _i32),
            pltpu.VMEM((CH,), _i32),
            pltpu.VMEM((CH,), _i32),
            pltpu.VMEM((CH,), _i32),
            pltpu.VMEM((KT,), _f32),
        ],
        compiler_params=pltpu.CompilerParams(needs_layout_passes=False),
    )


def _pass_a2(sk_h, inv_h, sc_out, skb, scb, invb):
    cid = lax.axis_index("c")
    t = lax.axis_index("s")
    pltpu.sync_copy(inv_h.at[cid], invb)

    def _chunk(c, carry):
        base = pl.multiple_of(t * PT + c * CH, 16)
        pltpu.sync_copy(sk_h.at[cid, pl.ds(base, CH)], skb)
        for j in range(CH // 16):
            s = pl.ds(j * 16, 16)
            scb[s] = plsc.load_gather(invb, [skb[s]])
        pltpu.sync_copy(scb, sc_out.at[cid, pl.ds(base, CH)])
        return carry
    lax.fori_loop(0, CT, _chunk, 0)


@functools.cache
def _pass_a2_call():
    return pl.kernel(
        _pass_a2,
        out_type=jax.ShapeDtypeStruct((2, EP), _f32),
        mesh=_mesh(),
        scratch_types=[
            pltpu.VMEM((CH,), _i32),
            pltpu.VMEM((CH,), _f32),
            pltpu.VMEM((KT,), _f32),
        ],
        compiler_params=pltpu.CompilerParams(needs_layout_passes=False),
    )


def _pass_b(h_flat, meta_h, acc_out,
            rows, cb, sb, gkb, dstb, scb, accs,
            gsem0, gsem1, ssem0, ssem1):
    cid = lax.axis_index("c")
    t = lax.axis_index("s")
    zeros16 = jnp.zeros((16,), _f32)
    gsems = (gsem0, gsem1)
    ssems = (ssem0, ssem1)

    def _zrow(r, carry):
        for j in range(8):
            rows[0, r, pl.ds(j * 16, 16)] = zeros16
        return carry
    lax.fori_loop(0, CH, _zrow, 0)
    for c in range(5):
        pltpu.sync_copy(rows.at[0], accs.at[pl.ds(t * 640 + c * CH, CH)])
    plsc.subcore_barrier()

    def _load_meta(c, slot):
        base = pl.multiple_of(t * PT + c * CH, CH)
        pltpu.sync_copy(meta_h.at[cid, 0, pl.ds(base, CH)], cb.at[slot])
        pltpu.sync_copy(meta_h.at[cid, 1, pl.ds(base, CH)], sb.at[slot])

    def _launch(slot):
        for j in range(CH // 16):
            s = pl.ds(j * 16, 16)
            cv = cb[slot, s]
            gkb[slot, s] = jnp.bitwise_and(cv, (1 << 18) - 1)
            dstb[slot, s] = lax.shift_right_logical(cv, 18)
            scb[slot, s] = plsc.bitcast(sb[slot, s], _f32)
        pltpu.async_copy(h_flat.at[gkb.at[slot]], rows.at[slot], gsems[slot])

    def _wait_scatter(slot):
        pltpu.make_async_copy(
            rows.at[slot], accs.at[dstb.at[slot]], ssems[slot]).wait()

    def _process(slot):
        pltpu.make_async_copy(
            h_flat.at[gkb.at[slot]], rows.at[slot], gsems[slot]).wait()

        def _edge(e, c2):
            for u in range(4):
                ee = e * 4 + u
                sp = plsc.load_gather(scb.at[slot],
                                      [jnp.full((16,), ee, _i32)])
                for j in range(8):
                    s = pl.ds(j * 16, 16)
                    rows[slot, ee, s] = rows[slot, ee, s] * sp
            return c2
        lax.fori_loop(0, CH // 4, _edge, 0)
        pltpu.async_copy(rows.at[slot], accs.at[dstb.at[slot]],
                         ssems[slot], add=True)

    _load_meta(0, 0)
    _launch(0)
    _load_meta(1, 1)
    _launch(1)
    _process(0)

    def _group(p, carry):
        for s in range(2):
            i = 2 * p + s
            _load_meta(i, s)
            _wait_scatter(s)
            _launch(s)
            _process(1 - s)
        return carry
    lax.fori_loop(1, CT // 2, _group, 0)

    _process(1)
    for slot in range(2):
        _wait_scatter(slot)
    plsc.subcore_barrier()

    pltpu.sync_copy(accs.at[pl.ds(t * 640, 640)],
                    acc_out.at[cid, pl.ds(t * 640, 640)])


@functools.cache
def _pass_b_call():
    return pl.kernel(
        _pass_b,
        out_type=jax.ShapeDtypeStruct((2, NP, HID), _f32),
        mesh=_mesh(),
        scratch_types=[
            pltpu.VMEM((2, CH, HID), _f32),
            pltpu.VMEM((2, CH), _i32),
            pltpu.VMEM((2, CH), _i32),
            pltpu.VMEM((2, CH), _i32),
            pltpu.VMEM((2, CH), _i32),
            pltpu.VMEM((2, CH), _f32),
            pltpu.VMEM_SHARED((NP, HID), _f32),
            pltpu.SemaphoreType.DMA,
            pltpu.SemaphoreType.DMA,
            pltpu.SemaphoreType.DMA,
            pltpu.SemaphoreType.DMA,
        ],
        compiler_params=pltpu.CompilerParams(needs_layout_passes=False),
    )


def _dense1_body(x_ref, basis_ref, comp_ref, root_ref, bias_ref, cnt_ref,
                 h_ref, o_ref, inv_ref):
    e = pl.program_id(0)
    r = pl.program_id(1)
    x = x_ref[0]
    w = comp_ref[e, r, 0] * basis_ref[0, 0]
    for b in range(1, R):
        w = w + comp_ref[e, r, b] * basis_ref[0, b]
    h_ref[0, 0] = jnp.dot(x, w, preferred_element_type=_f32)

    @pl.when(r == 0)
    def _():
        o_ref[0] = (jnp.dot(x, root_ref[0], preferred_element_type=_f32)
                    + bias_ref[0, 0])
        cnt = jnp.sum(cnt_ref[0], axis=0)
        inv_ref[0] = 1.0 / jnp.maximum(cnt, 1.0)


def _dense1(x, basis, comp, root, bias, cntp):
    return pl.pallas_call(
        _dense1_body,
        grid=(2, R),
        in_specs=[
            pl.BlockSpec((1, NP, EMBP), lambda e, r: (e, 0, 0)),
            pl.BlockSpec((1, R, EMBP, HID), lambda e, r: (e, 0, 0, 0)),
            pl.BlockSpec(memory_space=pltpu.SMEM),
            pl.BlockSpec((1, EMBP, HID), lambda e, r: (e, 0, 0)),
            pl.BlockSpec((1, 1, HID), lambda e, r: (e, 0, 0)),
            pl.BlockSpec((1, NT, KR, 128), lambda e, r: (e, 0, 0, 0)),
        ],
        out_specs=[
            pl.BlockSpec((1, 1, NP, HID), lambda e, r: (e, r, 0, 0)),
            pl.BlockSpec((1, NP, HID), lambda e, r: (e, 0, 0)),
            pl.BlockSpec((1, KR, 128), lambda e, r: (e, 0, 0)),
        ],
        out_shape=[
            jax.ShapeDtypeStruct((2, R, NP, HID), _f32),
            jax.ShapeDtypeStruct((2, NP, HID), _f32),
            jax.ShapeDtypeStruct((2, KR, 128), _f32),
        ],
    )(x, basis, comp, root, bias, cntp)


def _dense2_body(o1_ref, a1_ref, basis_ref, comp_ref, root_ref, bias_ref,
                 h_ref, o_ref):
    e = pl.program_id(0)
    r = pl.program_id(1)
    x = jnp.maximum(o1_ref[0] + a1_ref[0], 0.0)
    w = comp_ref[e, r, 0] * basis_ref[0, 0]
    for b in range(1, R):
        w = w + comp_ref[e, r, b] * basis_ref[0, b]
    h_ref[0, 0] = jnp.dot(x, w, preferred_element_type=_f32)

    @pl.when(r == 0)
    def _():
        o_ref[0] = (jnp.dot(x, root_ref[0], preferred_element_type=_f32)
                    + bias_ref[0, 0])


def _dense2(o1, a1, basis, comp, root, bias):
    return pl.pallas_call(
        _dense2_body,
        grid=(2, R),
        in_specs=[
            pl.BlockSpec((1, NP, HID), lambda e, r: (e, 0, 0)),
            pl.BlockSpec((1, NP, HID), lambda e, r: (e, 0, 0)),
            pl.BlockSpec((1, R, HID, HID), lambda e, r: (e, 0, 0, 0)),
            pl.BlockSpec(memory_space=pltpu.SMEM),
            pl.BlockSpec((1, HID, HID), lambda e, r: (e, 0, 0)),
            pl.BlockSpec((1, 1, HID), lambda e, r: (e, 0, 0)),
        ],
        out_specs=[
            pl.BlockSpec((1, 1, NP, HID), lambda e, r: (e, r, 0, 0)),
            pl.BlockSpec((1, NP, HID), lambda e, r: (e, 0, 0)),
        ],
        out_shape=[
            jax.ShapeDtypeStruct((2, R, NP, HID), _f32),
            jax.ShapeDtypeStruct((2, NP, HID), _f32),
        ],
    )(o1, a1, basis, comp, root, bias)


def _head_body(o2_ref, a2_ref, batch_ref, depth_ref, demb_ref, wbt_ref,
               bbil_ref, w1a_ref, w1b_ref, w1c_ref, w1d_ref, w1e_ref,
               b1_ref, w2_ref, b2_ref, out_ref):
    pooled = []
    for e in range(2):
        x3 = jnp.maximum(o2_ref[e] + a2_ref[e], 0.0)
        bt = batch_ref[e, 0]
        oh = (lax.broadcasted_iota(_i32, (G, NP), 0) == bt[None, :])
        oh = oh.astype(_f32)
        ssum = jnp.dot(oh, x3, preferred_element_type=_f32)
        n = jnp.sum(oh, axis=1, keepdims=True)
        pooled.append(ssum / jnp.maximum(n, 1.0))
    hs, hg = pooled

    tt = jnp.dot(hs, wbt_ref[...], preferred_element_type=_f32)
    cols = []
    for k in range(CROSS):
        seg = tt[:, k * HID:(k + 1) * HID] * hg
        cols.append(jnp.sum(seg, axis=1, keepdims=True))
    cross = jnp.concatenate(cols, axis=1) + bbil_ref[...]

    dint = depth_ref[...]
    dfeat = dint.astype(_f32)
    dmin = jnp.minimum(dint, VOC - 1)
    ohd = (lax.broadcasted_iota(_i32, (G, VOC), 1) == dmin).astype(_f32)
    demb = jnp.dot(ohd, demb_ref[...], preferred_element_type=_f32)

    h1 = jnp.maximum(
        jnp.dot(hs, w1a_ref[...], preferred_element_type=_f32)
        + jnp.dot(hg, w1b_ref[...], preferred_element_type=_f32)
        + jnp.dot(cross, w1c_ref[...], preferred_element_type=_f32)
        + jnp.dot(dfeat, w1d_ref[...], preferred_element_type=_f32)
        + jnp.dot(demb, w1e_ref[...], preferred_element_type=_f32)
        + b1_ref[...], 0.0)
    out_ref[...] = jnp.dot(h1, w2_ref[...], preferred_element_type=_f32) \
        + b2_ref[...]


def _head(o2, a2, batch3, depth2, depth_emb, wbt, bbil, w1a, w1b, w1c, w1d,
          w1e, b1, w2, b2):
    return pl.pallas_call(
        _head_body,
        out_shape=jax.ShapeDtypeStruct((G, 1), _f32),
    )(o2, a2, batch3, depth2, depth_emb, wbt, bbil, w1a, w1b, w1c, w1d,
      w1e, b1, w2, b2)


def _pad1(a, n_to, val):
    return jnp.concatenate(
        [a, jnp.full((n_to - a.shape[0],), val, a.dtype)])


def kernel(x_s, edge_index_s, edge_type_s, batch_s, x_g, edge_index_g,
           edge_type_g, batch_g, depth, emb_s, comp1_s, basis1_s, root1_s,
           bias1_s, comp2_s, basis2_s, root2_s, bias2_s, emb_g, comp1_g,
           basis1_g, root1_g, bias1_g, comp2_g, basis2_g, root2_g, bias2_g,
           W_bil, b_bil, depth_emb, W1, b1, W2, b2):
    i32 = _i32
    ids = jnp.stack([_pad1(x_s.astype(i32), NP, 0),
                     _pad1(x_g.astype(i32), NP, 0)])
    src = jnp.stack([_pad1(edge_index_s[0].astype(i32), EP, 0),
                     _pad1(edge_index_g[0].astype(i32), EP, 0)])
    dst = jnp.stack([_pad1(edge_index_s[1].astype(i32), EP, N),
                     _pad1(edge_index_g[1].astype(i32), EP, N)])
    et = jnp.stack([_pad1(edge_type_s.astype(i32), EP, 0),
                    _pad1(edge_type_g.astype(i32), EP, 0)])
    embf = jnp.concatenate([emb_s, emb_g], axis=0)
    embf = jnp.pad(embf, ((0, 0), (0, EMBP - EMB)))

    x_sg, gk, sk, cntp = _pass_a_call()(ids, src, dst, et, embf)

    basis1 = jnp.pad(jnp.stack([basis1_s, basis1_g]),
                     ((0, 0), (0, 0), (0, EMBP - EMB), (0, 0)))
    comp1 = jnp.stack([comp1_s, comp1_g])
    root1 = jnp.pad(jnp.stack([root1_s, root1_g]),
                    ((0, 0), (0, EMBP - EMB), (0, 0)))
    bias1 = jnp.stack([bias1_s, bias1_g])[:, None, :]
    h1, o1, inv = _dense1(x_sg, basis1, comp1, root1, bias1,
                          cntp.reshape(2, NT, KR, 128))
    inv2 = inv.reshape(2, KT)
    scale = _pass_a2_call()(sk, inv2)
    meta = jnp.stack(
        [gk, jax.lax.bitcast_convert_type(scale, _i32)], axis=1)
    acc1 = _pass_b_call()(h1.reshape(2 * KT, HID), meta)

    basis2 = jnp.stack([basis2_s, basis2_g])
    comp2 = jnp.stack([comp2_s, comp2_g])
    root2 = jnp.stack([root2_s, root2_g])
    bias2 = jnp.stack([bias2_s, bias2_g])[:, None, :]
    h2, o2 = _dense2(o1, acc1, basis2, comp2, root2, bias2)
    acc2 = _pass_b_call()(h2.reshape(2 * KT, HID), meta)

    batch3 = jnp.stack([_pad1(batch_s.astype(i32), NP, G),
                        _pad1(batch_g.astype(i32), NP, G)])[:, None, :]
    depth2 = depth.astype(i32)[:, None]
    wbt = W_bil.transpose(1, 0, 2).reshape(HID, CROSS * HID)
    bbil = b_bil[None, :]
    w1a = W1[:HID]
    w1b = W1[HID:2 * HID]
    w1c = W1[2 * HID:2 * HID + CROSS]
    w1d = W1[2 * HID + CROSS:2 * HID + CROSS + 1]
    w1e = W1[2 * HID + CROSS + 1:]
    out = _head(o2, acc2, batch3, depth2, depth_emb, wbt, bbil,
                w1a, w1b, w1c, w1d, w1e, b1[None, :], W2, b2[None, :])
    return out[:, 0]
